# baseline (speedup 1.0000x reference)
import jax
import jax.numpy as jnp
from jax.experimental import pallas as pl


def kernel(node_feat, edge_index, edge_feat):
    num_nodes = node_feat.shape[0]
    E = edge_index.shape[1]
    n_clusters = int(num_nodes * 0.5)

    cluster = jax.random.randint(jax.random.key(42), (num_nodes,), 0, n_clusters)

    src0 = jnp.take(cluster, edge_index[0])
    dst0 = jnp.take(cluster, edge_index[1])
    pair_key = src0 * n_clusters + dst0
    uniq, inv = jnp.unique(pair_key, size=E, fill_value=-1, return_inverse=True)
    inv = inv.reshape(-1)
    new_edge_feat = jax.ops.segment_sum(edge_feat, inv, num_segments=E)
    valid = uniq >= 0
    src = jnp.where(valid, uniq // n_clusters, -1)
    dst = jnp.where(valid, uniq % n_clusters, -1)
    keep = valid & (src != dst)
    new_edge_feat = new_edge_feat * keep[:, None].astype(new_edge_feat.dtype)
    src = jnp.where(keep, src, -1)
    dst = jnp.where(keep, dst, -1)

    old_nodes_idx = jnp.arange(num_nodes, dtype=cluster.dtype)
    new_dst_nodes = cluster + num_nodes
    inter_src = jnp.zeros(num_nodes * 2, dtype=cluster.dtype)
    inter_src = inter_src.at[0::2].set(old_nodes_idx).at[1::2].set(new_dst_nodes)
    inter_dst = jnp.zeros(num_nodes * 2, dtype=cluster.dtype)
    inter_dst = inter_dst.at[0::2].set(new_dst_nodes).at[1::2].set(old_nodes_idx)

    cluster_score = jnp.ones((n_clusters,), dtype=jnp.float32)
    return (src, dst, inter_src, inter_dst, cluster, new_edge_feat, cluster_score)


# speedup vs baseline: 1.5169x; 1.5169x over previous
import functools
import jax
import jax.numpy as jnp
from jax import lax
from jax.experimental import pallas as pl
from jax.experimental.pallas import tpu as pltpu, tpu_sc as plsc

E = 320000
N_NODES = 10000
N_CLUST = 5000
DSH = 13
BSH = 20
NBUK = 40
NW = 32
EPW = E // NW
K1CH = 2000
CHK = 1024
ESZ = E + NBUK * CHK + CHK
CHROWS = 98304
NWIN = 4
L = 16

MESH = plsc.VectorSubcoreMesh(core_axis_name="c", subcore_axis_name="s")
CP = pltpu.CompilerParams(needs_layout_passes=False, use_tc_tiling_on_sc=False)
I32 = jnp.int32


def _iota():
    return lax.iota(I32, L)


def _bucket_meta(ht, wid, pb_ref, tot_ref, off_ref):
    iota = _iota()
    bb = I32(0)
    zero = jnp.zeros((L,), I32)
    pbv = [zero, zero, zero]
    totv = [zero, zero, zero]
    offv = [zero, zero, zero]
    for b in range(NBUK):
        r0 = ht[pl.ds(b * 32, L)]
        r1 = ht[pl.ds(b * 32 + L, L)]
        tot = jnp.sum(r0, axis=0) + jnp.sum(r1, axis=0)
        lane = iota == (b % L)
        g = b // L
        if off_ref is not None:
            p0 = jnp.sum(jnp.where(iota < wid, r0, 0), axis=0)
            p1 = jnp.sum(jnp.where(iota + L < wid, r1, 0), axis=0)
            offv[g] = jnp.where(lane, bb + p0 + p1, offv[g])
        pbv[g] = jnp.where(lane, bb, pbv[g])
        totv[g] = jnp.where(lane, tot, totv[g])
        bb = bb + jnp.bitwise_and(tot + (CHK - 1), I32(-CHK))
    for g in range(3):
        pb_ref[pl.ds(g * L, L)] = pbv[g]
        tot_ref[pl.ds(g * L, L)] = totv[g]
        if off_ref is not None:
            off_ref[pl.ds(g * L, L)] = offv[g]
    return bb


def _sload(ref, idx):
    return ref[pl.ds(idx, L)][0]


def _popcount(v):
    c1 = I32(0x55555555)
    c2 = I32(0x33333333)
    c4 = I32(0x0F0F0F0F)
    v = v - jnp.bitwise_and(lax.shift_right_logical(v, 1), c1)
    v = jnp.bitwise_and(v, c2) + jnp.bitwise_and(lax.shift_right_logical(v, 2), c2)
    v = jnp.bitwise_and(v + lax.shift_right_logical(v, 4), c4)
    return lax.shift_right_logical(v * I32(0x01010101), 24)


@functools.partial(
    pl.kernel, mesh=MESH, compiler_params=CP,
    out_type=(jax.ShapeDtypeStruct((E,), I32),
              jax.ShapeDtypeStruct((1536,), I32)),
    scratch_types=[pltpu.VMEM((N_NODES,), I32),
                   pltpu.VMEM((K1CH,), I32), pltpu.VMEM((K1CH,), I32),
                   pltpu.VMEM((K1CH,), I32),
                   pltpu.VMEM((48,), I32), pltpu.VMEM((1, 48), I32),
                   pltpu.SemaphoreType.DMA],
)
def _k1(ei_hbm, clu_hbm, keys_hbm, hist_hbm, clu_v, sb, db, kb, hv, hidx, sem):
    cid = lax.axis_index("c")
    sid = lax.axis_index("s")
    wid = sid * 2 + cid
    iota = _iota()
    pltpu.sync_copy(clu_hbm, clu_v)
    for j in range(3):
        hv[pl.ds(j * L, L)] = jnp.zeros((L,), I32)

    def chunk(c, carry):
        base = wid * EPW + c * K1CH
        pltpu.sync_copy(ei_hbm.at[pl.ds(base, K1CH)], sb)
        pltpu.sync_copy(ei_hbm.at[pl.ds(E + base, K1CH)], db)
        for i in range(K1CH // L):
            u = sb[pl.ds(i * L, L)]
            v = db[pl.ds(i * L, L)]
            cs = plsc.load_gather(clu_v, [u])
            cd = plsc.load_gather(clu_v, [v])
            key = jnp.bitwise_or(lax.shift_left(cs, DSH), cd)
            kb[pl.ds(i * L, L)] = key
            b = lax.shift_right_logical(key, BSH)
            cnt, lastm = plsc.scan_count(b)
            old = plsc.load_gather(hv, [b], mask=lastm)
            plsc.store_scatter(hv, [b], old + cnt, mask=lastm)
        pltpu.sync_copy(kb, keys_hbm.at[pl.ds(base, K1CH)])
        return carry

    lax.fori_loop(0, EPW // K1CH, chunk, 0)
    for j in range(3):
        hidx[0, pl.ds(j * L, L)] = (iota + j * L) * 32 + wid
    pltpu.async_copy(hv, hist_hbm.at[hidx.at[0]], sem).wait()


@functools.partial(
    pl.kernel, mesh=MESH, compiler_params=CP,
    out_type=(jax.ShapeDtypeStruct((ESZ,), I32),
              jax.ShapeDtypeStruct((ESZ,), I32)),
    scratch_types=[pltpu.VMEM((1536,), I32),
                   pltpu.VMEM((64,), I32), pltpu.VMEM((64,), I32),
                   pltpu.VMEM((64,), I32),
                   pltpu.VMEM((K1CH,), I32), pltpu.VMEM((K1CH,), I32),
                   pltpu.VMEM((K1CH,), I32), pltpu.VMEM((25, 80), I32),
                   pltpu.SemaphoreType.DMA, pltpu.SemaphoreType.DMA],
)
def _k3(keys_hbm, hist_hbm, keyp_hbm, idp_hbm, ht, pb, tot, offv,
        keysb, kval, ival, sidx, sem, sem2):
    cid = lax.axis_index("c")
    sid = lax.axis_index("s")
    wid = sid * 2 + cid
    iota = _iota()
    pltpu.sync_copy(hist_hbm, ht)
    _bucket_meta(ht, wid, pb, tot, offv)

    def chunk(c, carry):
        base = wid * EPW + c * K1CH
        pltpu.sync_copy(keys_hbm.at[pl.ds(base, K1CH)], keysb)
        for i in range(K1CH // L):
            key = keysb[pl.ds(i * L, L)]
            b = lax.shift_right_logical(key, BSH)
            cnt, lastm = plsc.scan_count(b)
            basev = plsc.load_gather(offv, [b])
            plsc.store_scatter(offv, [b], basev + cnt, mask=lastm)
            sidx[i // 5, pl.ds((i % 5) * L, L)] = basev + cnt - 1
            kval[pl.ds(i * L, L)] = key
            ival[pl.ds(i * L, L)] = base + i * L + iota
        d1 = [pltpu.async_copy(kval.at[pl.ds(r * 80, 80)],
                               keyp_hbm.at[sidx.at[r]], sem) for r in range(25)]
        for d in d1:
            d.wait()
        d2 = [pltpu.async_copy(ival.at[pl.ds(r * 80, 80)],
                               idp_hbm.at[sidx.at[r]], sem2) for r in range(25)]
        for d in d2:
            d.wait()
        return carry

    lax.fori_loop(0, EPW // K1CH, chunk, 0)


NWORDS = 1 << (BSH - 5)


@functools.partial(
    pl.kernel, mesh=MESH, compiler_params=CP,
    out_type=(jax.ShapeDtypeStruct((ESZ,), I32),
              jax.ShapeDtypeStruct((768,), I32)),
    scratch_types=[pltpu.VMEM((1536,), I32),
                   pltpu.VMEM((64,), I32), pltpu.VMEM((64,), I32),
                   pltpu.VMEM((NWORDS,), I32), pltpu.VMEM((NWORDS,), I32),
                   pltpu.VMEM((CHK,), I32), pltpu.VMEM((CHK,), I32),
                   pltpu.VMEM((L,), I32), pltpu.VMEM((L,), I32),
                   pltpu.VMEM((CHK,), I32),
                   pltpu.SemaphoreType.DMA],
)
def _k4(keyp_hbm, hist_hbm, lrank_hbm, uc_hbm, ht, pb, tot,
        bm, wp, kb, lrb, scr, uv, zb, sem):
    cid = lax.axis_index("c")
    sid = lax.axis_index("s")
    iota = _iota()
    pltpu.sync_copy(hist_hbm, ht)
    _bucket_meta(ht, 0, pb, tot, None)
    for j in range(CHK // L):
        zb[pl.ds(j * L, L)] = jnp.zeros((L,), I32)

    def process_bucket(b):
        lb_n = _sload(tot, b)
        base = pl.multiple_of(_sload(pb, b), CHK)
        boff = lax.shift_left(b, BSH)
        nch = lax.shift_right_logical(lb_n + (CHK - 1), 10)

        def zchunk(i, carry):
            bm[pl.ds(i * L, L)] = jnp.zeros((L,), I32)
            return carry
        lax.fori_loop(0, NWORDS // L, zchunk, 0)

        def build(ch, carry):
            st = base + ch * CHK
            pltpu.sync_copy(keyp_hbm.at[pl.ds(st, CHK)], kb)
            for i in range(CHK // L):
                valid = (ch * CHK + i * L + iota) < lb_n
                kloc = kb[pl.ds(i * L, L)] - boff
                ks, _, vm = plsc.sort_key_val(kloc, iota, mask=valid)
                word = lax.shift_right_logical(ks, 5)
                bit = lax.shift_left(I32(1), jnp.bitwise_and(ks, 31))
                scr[...] = ks
                kprev = plsc.load_gather(scr, [jnp.maximum(iota - 1, 0)])
                firstk = ((iota == 0) | (kprev != ks)) & vm
                scr[...] = word
                wprev = plsc.load_gather(scr, [jnp.maximum(iota - 1, 0)])
                wnxt = plsc.load_gather(scr, [jnp.minimum(iota + 1, L - 1)])
                firstw = ((iota == 0) | (wprev != word)) & vm
                lastw = ((iota == L - 1) | (wnxt != word)) & vm
                contrib = jnp.where(firstk, bit, 0)
                cs = plsc.cumsum(contrib)
                spos = plsc.cummax(jnp.where(firstw, iota, 0))
                scr[...] = cs
                csm = plsc.load_gather(scr, [jnp.maximum(spos - 1, 0)])
                csprev = jnp.where(spos == 0, 0, csm)
                wordsum = cs - csprev
                old = plsc.load_gather(bm, [word], mask=lastw)
                plsc.store_scatter(bm, [word], jnp.bitwise_or(old, wordsum),
                                   mask=lastw)
            return carry
        lax.fori_loop(0, nch, build, 0)

        def prefix(i, carry):
            w32 = bm[pl.ds(i * L, L)]
            pc = _popcount(w32)
            cs = plsc.cumsum(pc)
            wp[pl.ds(i * L, L)] = cs - pc + carry
            return carry + jnp.sum(pc, axis=0)
        ucount = lax.fori_loop(0, NWORDS // L, prefix, I32(0))

        uv[...] = jnp.where(iota == 0, ucount, 0)
        pltpu.sync_copy(uv, uc_hbm.at[pl.ds(pl.multiple_of(b * L, L), L)])

        def rank(ch, carry):
            st = base + ch * CHK
            pltpu.sync_copy(keyp_hbm.at[pl.ds(st, CHK)], kb)
            for i in range(CHK // L):
                kloc = kb[pl.ds(i * L, L)] - boff
                kc = jnp.bitwise_and(kloc, (1 << BSH) - 1)
                word = lax.shift_right_logical(kc, 5)
                below = jnp.bitwise_and(
                    lax.shift_left(I32(1), jnp.bitwise_and(kc, 31)) - 1,
                    plsc.load_gather(bm, [word]))
                lrb[pl.ds(i * L, L)] = (plsc.load_gather(wp, [word])
                                        + _popcount(below))
            pltpu.sync_copy(lrb, lrank_hbm.at[pl.ds(st, CHK)])
            return carry
        lax.fori_loop(0, nch, rank, 0)

    process_bucket(20 * cid + sid)

    @pl.when(sid < 4)
    def _():
        process_bucket(20 * cid + L + sid)


@functools.partial(
    pl.kernel, mesh=MESH, compiler_params=CP,
    out_type=(jax.ShapeDtypeStruct((E + L,), I32),
              jax.ShapeDtypeStruct((E + L,), I32),
              jax.ShapeDtypeStruct((E, L), jnp.float32)),
    scratch_types=[pltpu.VMEM((1536,), I32), pltpu.VMEM((768,), I32),
                   pltpu.VMEM((64,), I32), pltpu.VMEM((64,), I32),
                   pltpu.VMEM((64,), I32),
                   pltpu.VMEM((CHK,), I32), pltpu.VMEM((CHK,), I32),
                   pltpu.VMEM((CHK,), I32),
                   pltpu.VMEM((CHK,), I32), pltpu.VMEM((CHK,), I32),
                   pltpu.VMEM((8, 128), I32),
                   pltpu.VMEM((1, 128), I32), pltpu.VMEM((1, 128), I32),
                   pltpu.VMEM((128, L), jnp.float32),
                   pltpu.VMEM((112, L), jnp.float32),
                   pltpu.VMEM((64,), I32), pltpu.VMEM((1, 64), I32),
                   pltpu.VMEM_SHARED((CHROWS + 32, L), jnp.float32),
                   pltpu.SemaphoreType.DMA, pltpu.SemaphoreType.DMA,
                   pltpu.SemaphoreType.DMA, pltpu.SemaphoreType.DMA],
)
def _k5(keyp_hbm, idp_hbm, lrank_hbm, hist_hbm, uc_hbm, ef_hbm,
        src_hbm, dst_hbm, nef_hbm,
        ht, ucv, pb, tot, gbv, kb, lb, ib, svb, dvb,
        ridx, ridxg, gidxg, frows, zbuf, neg64, hidx2, acc,
        sem, sem2, sem3, sem4):
    cid = lax.axis_index("c")
    sid = lax.axis_index("s")
    wid = sid * 2 + cid
    iota = _iota()
    pltpu.sync_copy(hist_hbm, ht)
    _bucket_meta(ht, 0, pb, tot, None)
    pltpu.sync_copy(uc_hbm, ucv)
    run = I32(0)
    zero = jnp.zeros((L,), I32)
    gvv = [zero, zero, zero]
    for b in range(NBUK):
        gvv[b // L] = jnp.where(iota == (b % L), run, gvv[b // L])
        run = run + _sload(ucv, b * L)
    gvv[NBUK // L] = jnp.where(iota == (NBUK % L), run, gvv[NBUK // L])
    for g in range(3):
        gbv[pl.ds(g * L, L)] = gvv[g]
    nu = run
    for j in range(4):
        neg64[pl.ds(j * L, L)] = jnp.full((L,), -1, I32)
    for r in range(112):
        zbuf[r, :] = jnp.zeros((L,), jnp.float32)

    def stage_a(b):
        lb_n = _sload(tot, b)
        base = pl.multiple_of(_sload(pb, b), CHK)
        gb = _sload(gbv, b)
        nch = lax.shift_right_logical(lb_n + (CHK - 1), 10)

        def chunk(ch, carry):
            st = base + ch * CHK
            pltpu.sync_copy(keyp_hbm.at[pl.ds(st, CHK)], kb)
            pltpu.sync_copy(lrank_hbm.at[pl.ds(st, CHK)], lb)
            for i in range(CHK // L):
                valid = (ch * CHK + i * L + iota) < lb_n
                key = kb[pl.ds(i * L, L)]
                s = lax.shift_right_logical(key, DSH)
                d = jnp.bitwise_and(key, (1 << DSH) - 1)
                gr = gb + lb[pl.ds(i * L, L)]
                selfm = s == d
                svb[pl.ds(i * L, L)] = jnp.where(selfm, -1, s)
                dvb[pl.ds(i * L, L)] = jnp.where(selfm, -1, d)
                ridx[i // 8, pl.ds((i % 8) * L, L)] = jnp.where(valid, gr,
                                                               E + iota)
            ds1 = [pltpu.async_copy(svb.at[pl.ds(g * 128, 128)],
                                    src_hbm.at[ridx.at[g]], sem)
                   for g in range(8)]
            for d0 in ds1:
                d0.wait()
            ds2 = [pltpu.async_copy(dvb.at[pl.ds(g * 128, 128)],
                                    dst_hbm.at[ridx.at[g]], sem2)
                   for g in range(8)]
            for d0 in ds2:
                d0.wait()
            return carry
        lax.fori_loop(0, nch, chunk, 0)

    stage_a(20 * cid + sid)

    @pl.when(sid < 4)
    def _():
        stage_a(20 * cid + L + sid)

    @pl.when(wid == 0)
    def _():
        nuceil = jnp.minimum(jnp.bitwise_and(nu + 63, I32(-64)), I32(E))
        for j in range(4):
            off = j * L + iota
            hidx2[0, pl.ds(j * L, L)] = jnp.where(nu + off < nuceil, nu + off,
                                                  E + iota)
        pltpu.async_copy(neg64, src_hbm.at[hidx2.at[0]], sem).wait()
        pltpu.async_copy(neg64, dst_hbm.at[hidx2.at[0]], sem2).wait()

    def tail(t, carry):
        m = wid + NW * t
        mstart = pl.multiple_of(m * 64, 64)

        @pl.when((mstart >= nu) & (m < E // 64))
        def _():
            pltpu.sync_copy(neg64, src_hbm.at[pl.ds(mstart, 64)])
            pltpu.sync_copy(neg64, dst_hbm.at[pl.ds(mstart, 64)])
        return carry
    lax.fori_loop(0, E // 64 // NW + 1, tail, 0)

    scstart = _sload(gbv, 20 * cid)
    scend = jnp.where(cid == 0, _sload(gbv, 20), I32(E))

    def accum_bucket(b, win0):
        lb_n = _sload(tot, b)
        base = pl.multiple_of(_sload(pb, b), CHK)
        gb = _sload(gbv, b)
        nch = lax.shift_right_logical(lb_n + (CHK - 1), 10)

        def chunk(ch, carry):
            st = base + ch * CHK
            pltpu.sync_copy(keyp_hbm.at[pl.ds(st, CHK)], kb)
            pltpu.sync_copy(lrank_hbm.at[pl.ds(st, CHK)], lb)
            pltpu.sync_copy(idp_hbm.at[pl.ds(st, CHK)], ib)
            for g in range(8):
                ns = I32(0)
                for i2 in range(8):
                    i = g * 8 + i2
                    valid = (ch * CHK + i * L + iota) < lb_n
                    key = kb[pl.ds(i * L, L)]
                    s = lax.shift_right_logical(key, DSH)
                    d = jnp.bitwise_and(key, (1 << DSH) - 1)
                    gr = gb + lb[pl.ds(i * L, L)]
                    sel = (valid & (s != d) & (gr >= win0)
                           & (gr < win0 + CHROWS))
                    ridxg[0, pl.ds(i2 * L, L)] = jnp.where(sel, gr - win0,
                                                           CHROWS + iota)
                    gidxg[0, pl.ds(i2 * L, L)] = jnp.where(sel, ib[pl.ds(i * L, L)], 0)
                    ns = ns + jnp.sum(sel.astype(I32), axis=0)

                @pl.when(ns > 0)
                def _():
                    pltpu.async_copy(ef_hbm.at[gidxg.at[0]], frows, sem3).wait()
                    pltpu.async_copy(frows, acc.at[ridxg.at[0]], sem4,
                                     add=True).wait()
            return carry
        lax.fori_loop(0, nch, chunk, 0)

    for win in range(NWIN):
        win0 = scstart + win * CHROWS
        active = win0 < scend
        limit = jnp.minimum(win0 + CHROWS, scend)

        @pl.when(active)
        def _():
            def zrow(t, carry):
                j = sid + L * t

                @pl.when(j < (CHROWS + 32) // 112)
                def _():
                    pltpu.sync_copy(zbuf, acc.at[pl.ds(j * 112, 112)])
                return carry
            lax.fori_loop(0, (CHROWS + 32) // 112 // L + 1, zrow, 0)
        plsc.subcore_barrier()

        @pl.when(active)
        def _():
            accum_bucket(20 * cid + sid, win0)

            @pl.when(sid < 4)
            def _():
                accum_bucket(20 * cid + L + sid, win0)
        plsc.subcore_barrier()

        @pl.when(active)
        def _():
            nfull = lax.shift_right_logical(limit - win0, 9)

            def flush(t, carry):
                j = sid + L * t

                @pl.when(j < nfull)
                def _():
                    pltpu.sync_copy(acc.at[pl.ds(j * 512, 512)],
                                    nef_hbm.at[pl.ds(win0 + j * 512, 512)])
                return carry
            lax.fori_loop(0, CHROWS // 512 // L, flush, 0)

            @pl.when(sid == 0)
            def _():
                r1 = lax.shift_left(nfull, 9)
                n16 = lax.shift_right_logical(limit - win0 - r1, 4)

                def f16(t, carry):
                    @pl.when(t < n16)
                    def _():
                        pltpu.sync_copy(
                            acc.at[pl.ds(r1 + t * L, L)],
                            nef_hbm.at[pl.ds(win0 + r1 + t * L, L)])
                    return carry
                lax.fori_loop(0, 32, f16, 0)
                r2 = r1 + lax.shift_left(n16, 4)
                nrem = limit - win0 - r2

                def f1(t, carry):
                    @pl.when(t < nrem)
                    def _():
                        pltpu.sync_copy(acc.at[pl.ds(r2 + t, 1)],
                                        nef_hbm.at[pl.ds(win0 + r2 + t, 1)])
                    return carry
                lax.fori_loop(0, L, f1, 0)
        plsc.subcore_barrier()


def kernel(node_feat, edge_index, edge_feat):
    num_nodes = node_feat.shape[0]
    cluster = jax.random.randint(jax.random.key(42), (num_nodes,), 0, N_CLUST)
    cluster = cluster.astype(jnp.int32)

    ei_flat = edge_index.reshape(-1).astype(jnp.int32)
    keys1, hist = _k1(ei_flat, cluster)
    keyp, idp = _k3(keys1, hist)
    lrank, ucounts = _k4(keyp, hist)
    srcp, dstp, nef = _k5(keyp, idp, lrank, hist, ucounts, edge_feat)
    src = srcp[:E]
    dst = dstp[:E]

    old_nodes_idx = jnp.arange(num_nodes, dtype=cluster.dtype)
    new_dst_nodes = cluster + num_nodes
    inter_src = jnp.zeros(num_nodes * 2, dtype=cluster.dtype)
    inter_src = inter_src.at[0::2].set(old_nodes_idx).at[1::2].set(new_dst_nodes)
    inter_dst = jnp.zeros(num_nodes * 2, dtype=cluster.dtype)
    inter_dst = inter_dst.at[0::2].set(new_dst_nodes).at[1::2].set(old_nodes_idx)
    cluster_score = jnp.ones((N_CLUST,), dtype=jnp.float32)
    return (src, dst, inter_src, inter_dst, cluster, nef, cluster_score)


# speedup vs baseline: 1.6050x; 1.0581x over previous
import functools
import jax
import jax.numpy as jnp
from jax import lax
from jax.experimental import pallas as pl
from jax.experimental.pallas import tpu as pltpu, tpu_sc as plsc

E = 320000
N_NODES = 10000
N_CLUST = 5000
DSH = 13
BSH = 20
NBUK = 40
NW = 32
EPW = E // NW
K1CH = 2000
CHK = 1024
ESZ = E + NBUK * CHK + CHK
CHROWS = 98304
NWIN = 4
L = 16

MESH = plsc.VectorSubcoreMesh(core_axis_name="c", subcore_axis_name="s")
CP = pltpu.CompilerParams(needs_layout_passes=False, use_tc_tiling_on_sc=False)
I32 = jnp.int32


def _iota():
    return lax.iota(I32, L)


def _bucket_meta(ht, wid, pb_ref, tot_ref, off_ref):
    iota = _iota()
    bb = I32(0)
    zero = jnp.zeros((L,), I32)
    pbv = [zero, zero, zero]
    totv = [zero, zero, zero]
    offv = [zero, zero, zero]
    for b in range(NBUK):
        r0 = ht[pl.ds(b * 32, L)]
        r1 = ht[pl.ds(b * 32 + L, L)]
        tot = jnp.sum(r0, axis=0) + jnp.sum(r1, axis=0)
        lane = iota == (b % L)
        g = b // L
        if off_ref is not None:
            p0 = jnp.sum(jnp.where(iota < wid, r0, 0), axis=0)
            p1 = jnp.sum(jnp.where(iota + L < wid, r1, 0), axis=0)
            offv[g] = jnp.where(lane, bb + p0 + p1, offv[g])
        pbv[g] = jnp.where(lane, bb, pbv[g])
        totv[g] = jnp.where(lane, tot, totv[g])
        bb = bb + jnp.bitwise_and(tot + (CHK - 1), I32(-CHK))
    for g in range(3):
        pb_ref[pl.ds(g * L, L)] = pbv[g]
        tot_ref[pl.ds(g * L, L)] = totv[g]
        if off_ref is not None:
            off_ref[pl.ds(g * L, L)] = offv[g]
    return bb


def _sload(ref, idx):
    return ref[pl.ds(idx, L)][0]


def _popcount(v):
    c1 = I32(0x55555555)
    c2 = I32(0x33333333)
    c4 = I32(0x0F0F0F0F)
    v = v - jnp.bitwise_and(lax.shift_right_logical(v, 1), c1)
    v = jnp.bitwise_and(v, c2) + jnp.bitwise_and(lax.shift_right_logical(v, 2), c2)
    v = jnp.bitwise_and(v + lax.shift_right_logical(v, 4), c4)
    return lax.shift_right_logical(v * I32(0x01010101), 24)


@functools.partial(
    pl.kernel, mesh=MESH, compiler_params=CP,
    out_type=(jax.ShapeDtypeStruct((E,), I32),
              jax.ShapeDtypeStruct((1536,), I32)),
    scratch_types=[pltpu.VMEM((N_NODES,), I32),
                   pltpu.VMEM((K1CH,), I32), pltpu.VMEM((K1CH,), I32),
                   pltpu.VMEM((K1CH,), I32),
                   pltpu.VMEM((48,), I32), pltpu.VMEM((1, 48), I32),
                   pltpu.SemaphoreType.DMA],
)
def _k1(ei_hbm, clu_hbm, keys_hbm, hist_hbm, clu_v, sb, db, kb, hv, hidx, sem):
    cid = lax.axis_index("c")
    sid = lax.axis_index("s")
    wid = sid * 2 + cid
    iota = _iota()
    pltpu.sync_copy(clu_hbm, clu_v)
    for j in range(3):
        hv[pl.ds(j * L, L)] = jnp.zeros((L,), I32)

    def chunk(c, carry):
        base = wid * EPW + c * K1CH
        pltpu.sync_copy(ei_hbm.at[pl.ds(base, K1CH)], sb)
        pltpu.sync_copy(ei_hbm.at[pl.ds(E + base, K1CH)], db)
        for i in range(K1CH // L):
            u = sb[pl.ds(i * L, L)]
            v = db[pl.ds(i * L, L)]
            cs = plsc.load_gather(clu_v, [u])
            cd = plsc.load_gather(clu_v, [v])
            key = jnp.bitwise_or(lax.shift_left(cs, DSH), cd)
            kb[pl.ds(i * L, L)] = key
            b = lax.shift_right_logical(key, BSH)
            cnt, lastm = plsc.scan_count(b)
            old = plsc.load_gather(hv, [b], mask=lastm)
            plsc.store_scatter(hv, [b], old + cnt, mask=lastm)
        pltpu.sync_copy(kb, keys_hbm.at[pl.ds(base, K1CH)])
        return carry

    lax.fori_loop(0, EPW // K1CH, chunk, 0)
    for j in range(3):
        hidx[0, pl.ds(j * L, L)] = (iota + j * L) * 32 + wid
    pltpu.async_copy(hv, hist_hbm.at[hidx.at[0]], sem).wait()


@functools.partial(
    pl.kernel, mesh=MESH, compiler_params=CP,
    out_type=(jax.ShapeDtypeStruct((ESZ,), I32),
              jax.ShapeDtypeStruct((ESZ,), I32)),
    scratch_types=[pltpu.VMEM((1536,), I32),
                   pltpu.VMEM((64,), I32), pltpu.VMEM((64,), I32),
                   pltpu.VMEM((64,), I32),
                   pltpu.VMEM((K1CH,), I32), pltpu.VMEM((K1CH,), I32),
                   pltpu.VMEM((K1CH,), I32), pltpu.VMEM((25, 80), I32),
                   pltpu.SemaphoreType.DMA, pltpu.SemaphoreType.DMA],
)
def _k3(keys_hbm, hist_hbm, keyp_hbm, idp_hbm, ht, pb, tot, offv,
        keysb, kval, ival, sidx, sem, sem2):
    cid = lax.axis_index("c")
    sid = lax.axis_index("s")
    wid = sid * 2 + cid
    iota = _iota()
    pltpu.sync_copy(hist_hbm, ht)
    _bucket_meta(ht, wid, pb, tot, offv)

    def chunk(c, carry):
        base = wid * EPW + c * K1CH
        pltpu.sync_copy(keys_hbm.at[pl.ds(base, K1CH)], keysb)
        for i in range(K1CH // L):
            key = keysb[pl.ds(i * L, L)]
            b = lax.shift_right_logical(key, BSH)
            cnt, lastm = plsc.scan_count(b)
            basev = plsc.load_gather(offv, [b])
            plsc.store_scatter(offv, [b], basev + cnt, mask=lastm)
            sidx[i // 5, pl.ds((i % 5) * L, L)] = basev + cnt - 1
            kval[pl.ds(i * L, L)] = key
            ival[pl.ds(i * L, L)] = base + i * L + iota
        d1 = [pltpu.async_copy(kval.at[pl.ds(r * 80, 80)],
                               keyp_hbm.at[sidx.at[r]], sem) for r in range(25)]
        d2 = [pltpu.async_copy(ival.at[pl.ds(r * 80, 80)],
                               idp_hbm.at[sidx.at[r]], sem2) for r in range(25)]
        for d in d1 + d2:
            d.wait()
        return carry

    lax.fori_loop(0, EPW // K1CH, chunk, 0)


NWORDS = 1 << (BSH - 5)


@functools.partial(
    pl.kernel, mesh=MESH, compiler_params=CP,
    out_type=(jax.ShapeDtypeStruct((ESZ,), I32),
              jax.ShapeDtypeStruct((768,), I32)),
    scratch_types=[pltpu.VMEM((1536,), I32),
                   pltpu.VMEM((64,), I32), pltpu.VMEM((64,), I32),
                   pltpu.VMEM((NWORDS,), I32), pltpu.VMEM((NWORDS,), I32),
                   pltpu.VMEM((CHK,), I32), pltpu.VMEM((CHK,), I32),
                   pltpu.VMEM((L,), I32), pltpu.VMEM((L,), I32),
                   pltpu.VMEM((CHK,), I32),
                   pltpu.SemaphoreType.DMA],
)
def _k4(keyp_hbm, hist_hbm, lrank_hbm, uc_hbm, ht, pb, tot,
        bm, wp, kb, lrb, scr, uv, zb, sem):
    cid = lax.axis_index("c")
    sid = lax.axis_index("s")
    iota = _iota()
    pltpu.sync_copy(hist_hbm, ht)
    _bucket_meta(ht, 0, pb, tot, None)
    for j in range(CHK // L):
        zb[pl.ds(j * L, L)] = jnp.zeros((L,), I32)

    def process_bucket(b):
        lb_n = _sload(tot, b)
        base = pl.multiple_of(_sload(pb, b), CHK)
        boff = lax.shift_left(b, BSH)
        nch = lax.shift_right_logical(lb_n + (CHK - 1), 10)

        def zchunk(i, carry):
            bm[pl.ds(i * L, L)] = jnp.zeros((L,), I32)
            return carry
        lax.fori_loop(0, NWORDS // L, zchunk, 0)

        def build(ch, carry):
            st = base + ch * CHK
            pltpu.sync_copy(keyp_hbm.at[pl.ds(st, CHK)], kb)
            for i in range(CHK // L):
                valid = (ch * CHK + i * L + iota) < lb_n
                kloc = kb[pl.ds(i * L, L)] - boff
                ks, _, vm = plsc.sort_key_val(kloc, iota, mask=valid)
                word = lax.shift_right_logical(ks, 5)
                bit = lax.shift_left(I32(1), jnp.bitwise_and(ks, 31))
                scr[...] = ks
                kprev = plsc.load_gather(scr, [jnp.maximum(iota - 1, 0)])
                firstk = ((iota == 0) | (kprev != ks)) & vm
                scr[...] = word
                wprev = plsc.load_gather(scr, [jnp.maximum(iota - 1, 0)])
                wnxt = plsc.load_gather(scr, [jnp.minimum(iota + 1, L - 1)])
                firstw = ((iota == 0) | (wprev != word)) & vm
                lastw = ((iota == L - 1) | (wnxt != word)) & vm
                contrib = jnp.where(firstk, bit, 0)
                cs = plsc.cumsum(contrib)
                spos = plsc.cummax(jnp.where(firstw, iota, 0))
                scr[...] = cs
                csm = plsc.load_gather(scr, [jnp.maximum(spos - 1, 0)])
                csprev = jnp.where(spos == 0, 0, csm)
                wordsum = cs - csprev
                old = plsc.load_gather(bm, [word], mask=lastw)
                plsc.store_scatter(bm, [word], jnp.bitwise_or(old, wordsum),
                                   mask=lastw)
            return carry
        lax.fori_loop(0, nch, build, 0)

        def prefix(i, carry):
            w32 = bm[pl.ds(i * L, L)]
            pc = _popcount(w32)
            cs = plsc.cumsum(pc)
            wp[pl.ds(i * L, L)] = cs - pc + carry
            return carry + jnp.sum(pc, axis=0)
        ucount = lax.fori_loop(0, NWORDS // L, prefix, I32(0))

        uv[...] = jnp.where(iota == 0, ucount, 0)
        pltpu.sync_copy(uv, uc_hbm.at[pl.ds(pl.multiple_of(b * L, L), L)])

        def rank(ch, carry):
            st = base + ch * CHK
            pltpu.sync_copy(keyp_hbm.at[pl.ds(st, CHK)], kb)
            for i in range(CHK // L):
                kloc = kb[pl.ds(i * L, L)] - boff
                kc = jnp.bitwise_and(kloc, (1 << BSH) - 1)
                word = lax.shift_right_logical(kc, 5)
                below = jnp.bitwise_and(
                    lax.shift_left(I32(1), jnp.bitwise_and(kc, 31)) - 1,
                    plsc.load_gather(bm, [word]))
                lrb[pl.ds(i * L, L)] = (plsc.load_gather(wp, [word])
                                        + _popcount(below))
            pltpu.sync_copy(lrb, lrank_hbm.at[pl.ds(st, CHK)])
            return carry
        lax.fori_loop(0, nch, rank, 0)

    process_bucket(20 * cid + sid)

    @pl.when(sid < 4)
    def _():
        process_bucket(20 * cid + L + sid)


@functools.partial(
    pl.kernel, mesh=MESH, compiler_params=CP,
    out_type=(jax.ShapeDtypeStruct((E + L,), I32),
              jax.ShapeDtypeStruct((E + L,), I32),
              jax.ShapeDtypeStruct((E, L), jnp.float32)),
    scratch_types=[pltpu.VMEM((1536,), I32), pltpu.VMEM((768,), I32),
                   pltpu.VMEM((64,), I32), pltpu.VMEM((64,), I32),
                   pltpu.VMEM((64,), I32),
                   pltpu.VMEM((CHK,), I32), pltpu.VMEM((CHK,), I32),
                   pltpu.VMEM((CHK,), I32),
                   pltpu.VMEM((CHK,), I32), pltpu.VMEM((CHK,), I32),
                   pltpu.VMEM((8, 128), I32), pltpu.VMEM((8, 128), I32),
                   pltpu.VMEM((1024, L), jnp.float32),
                   pltpu.VMEM((112, L), jnp.float32),
                   pltpu.VMEM((64,), I32), pltpu.VMEM((1, 64), I32),
                   pltpu.VMEM_SHARED((CHROWS + 32, L), jnp.float32),
                   pltpu.SemaphoreType.DMA, pltpu.SemaphoreType.DMA,
                   pltpu.SemaphoreType.DMA, pltpu.SemaphoreType.DMA],
)
def _k5(keyp_hbm, idp_hbm, lrank_hbm, hist_hbm, uc_hbm, ef_hbm,
        src_hbm, dst_hbm, nef_hbm,
        ht, ucv, pb, tot, gbv, kb, lb, ib, svb, dvb,
        ridx, gidx2, frows, zbuf, neg64, hidx2, acc,
        sem, sem2, sem3, sem4):
    cid = lax.axis_index("c")
    sid = lax.axis_index("s")
    wid = sid * 2 + cid
    iota = _iota()
    pltpu.sync_copy(hist_hbm, ht)
    _bucket_meta(ht, 0, pb, tot, None)
    pltpu.sync_copy(uc_hbm, ucv)
    run = I32(0)
    zero = jnp.zeros((L,), I32)
    gvv = [zero, zero, zero]
    for b in range(NBUK):
        gvv[b // L] = jnp.where(iota == (b % L), run, gvv[b // L])
        run = run + _sload(ucv, b * L)
    gvv[NBUK // L] = jnp.where(iota == (NBUK % L), run, gvv[NBUK // L])
    for g in range(3):
        gbv[pl.ds(g * L, L)] = gvv[g]
    nu = run
    for j in range(4):
        neg64[pl.ds(j * L, L)] = jnp.full((L,), -1, I32)
    for r in range(112):
        zbuf[r, :] = jnp.zeros((L,), jnp.float32)

    def stage_a(b):
        lb_n = _sload(tot, b)
        base = pl.multiple_of(_sload(pb, b), CHK)
        gb = _sload(gbv, b)
        nch = lax.shift_right_logical(lb_n + (CHK - 1), 10)

        def chunk(ch, carry):
            st = base + ch * CHK
            pltpu.sync_copy(keyp_hbm.at[pl.ds(st, CHK)], kb)
            pltpu.sync_copy(lrank_hbm.at[pl.ds(st, CHK)], lb)
            for i in range(CHK // L):
                valid = (ch * CHK + i * L + iota) < lb_n
                key = kb[pl.ds(i * L, L)]
                s = lax.shift_right_logical(key, DSH)
                d = jnp.bitwise_and(key, (1 << DSH) - 1)
                gr = gb + lb[pl.ds(i * L, L)]
                selfm = s == d
                svb[pl.ds(i * L, L)] = jnp.where(selfm, -1, s)
                dvb[pl.ds(i * L, L)] = jnp.where(selfm, -1, d)
                ridx[i // 8, pl.ds((i % 8) * L, L)] = jnp.where(valid, gr,
                                                               E + iota)
            ds1 = [pltpu.async_copy(svb.at[pl.ds(g * 128, 128)],
                                    src_hbm.at[ridx.at[g]], sem)
                   for g in range(8)]
            ds2 = [pltpu.async_copy(dvb.at[pl.ds(g * 128, 128)],
                                    dst_hbm.at[ridx.at[g]], sem2)
                   for g in range(8)]
            for d0 in ds1 + ds2:
                d0.wait()
            return carry
        lax.fori_loop(0, nch, chunk, 0)

    stage_a(20 * cid + sid)

    @pl.when(sid < 4)
    def _():
        stage_a(20 * cid + L + sid)

    @pl.when(wid == 0)
    def _():
        nuceil = jnp.minimum(jnp.bitwise_and(nu + 63, I32(-64)), I32(E))
        for j in range(4):
            off = j * L + iota
            hidx2[0, pl.ds(j * L, L)] = jnp.where(nu + off < nuceil, nu + off,
                                                  E + iota)
        pltpu.async_copy(neg64, src_hbm.at[hidx2.at[0]], sem).wait()
        pltpu.async_copy(neg64, dst_hbm.at[hidx2.at[0]], sem2).wait()

    def tail(t, carry):
        m = wid + NW * t
        mstart = pl.multiple_of(m * 64, 64)

        @pl.when((mstart >= nu) & (m < E // 64))
        def _():
            pltpu.sync_copy(neg64, src_hbm.at[pl.ds(mstart, 64)])
            pltpu.sync_copy(neg64, dst_hbm.at[pl.ds(mstart, 64)])
        return carry
    lax.fori_loop(0, E // 64 // NW + 1, tail, 0)

    scstart = _sload(gbv, 20 * cid)
    scend = jnp.where(cid == 0, _sload(gbv, 20), I32(E))

    def accum_bucket(b, win0):
        lb_n = _sload(tot, b)
        base = pl.multiple_of(_sload(pb, b), CHK)
        gb = _sload(gbv, b)
        nch = lax.shift_right_logical(lb_n + (CHK - 1), 10)

        def chunk(ch, carry):
            st = base + ch * CHK
            pltpu.sync_copy(keyp_hbm.at[pl.ds(st, CHK)], kb)
            pltpu.sync_copy(lrank_hbm.at[pl.ds(st, CHK)], lb)
            pltpu.sync_copy(idp_hbm.at[pl.ds(st, CHK)], ib)
            nsg = []
            for g in range(8):
                ns = I32(0)
                for i2 in range(8):
                    i = g * 8 + i2
                    valid = (ch * CHK + i * L + iota) < lb_n
                    key = kb[pl.ds(i * L, L)]
                    s = lax.shift_right_logical(key, DSH)
                    d = jnp.bitwise_and(key, (1 << DSH) - 1)
                    gr = gb + lb[pl.ds(i * L, L)]
                    sel = (valid & (s != d) & (gr >= win0)
                           & (gr < win0 + CHROWS))
                    ridx[g, pl.ds(i2 * L, L)] = jnp.where(sel, gr - win0,
                                                          CHROWS + iota)
                    gidx2[g, pl.ds(i2 * L, L)] = jnp.where(
                        sel, ib[pl.ds(i * L, L)], iota)
                    ns = ns + jnp.sum(sel.astype(I32), axis=0)
                nsg.append(ns)
            for g in range(8):
                @pl.when(nsg[g] > 0)
                def _():
                    pltpu.async_copy(ef_hbm.at[gidx2.at[g]],
                                     frows.at[pl.ds(g * 128, 128)], sem3).wait()
                    pltpu.async_copy(frows.at[pl.ds(g * 128, 128)],
                                     acc.at[ridx.at[g]], sem4, add=True).wait()
            return carry
        lax.fori_loop(0, nch, chunk, 0)

    for win in range(NWIN):
        win0 = scstart + win * CHROWS
        active = win0 < scend
        limit = jnp.minimum(win0 + CHROWS, scend)

        @pl.when(active)
        def _():
            nz = (CHROWS + 32) // 112
            dz = []
            for t in range(nz // L + 1):
                j = sid + L * t
                if t < nz // L:
                    dz.append(pltpu.async_copy(
                        zbuf, acc.at[pl.ds(pl.multiple_of(j * 112, 16), 112)],
                        sem3))
            jlast = sid + L * (nz // L)

            @pl.when(jlast < nz)
            def _():
                pltpu.sync_copy(zbuf,
                                acc.at[pl.ds(pl.multiple_of(jlast * 112, 16), 112)])
            for d0 in dz:
                d0.wait()
        plsc.subcore_barrier()

        @pl.when(active)
        def _():
            accum_bucket(20 * cid + sid, win0)

            @pl.when(sid < 4)
            def _():
                accum_bucket(20 * cid + L + sid, win0)
        plsc.subcore_barrier()

        @pl.when(active)
        def _():
            nfull = lax.shift_right_logical(limit - win0, 9)

            def flush(t, carry):
                j = sid + L * t

                @pl.when(j < nfull)
                def _():
                    pltpu.sync_copy(acc.at[pl.ds(j * 512, 512)],
                                    nef_hbm.at[pl.ds(win0 + j * 512, 512)])
                return carry
            lax.fori_loop(0, CHROWS // 512 // L, flush, 0)

            @pl.when(sid == 0)
            def _():
                r1 = lax.shift_left(nfull, 9)
                n16 = lax.shift_right_logical(limit - win0 - r1, 4)

                def f16(t, carry):
                    @pl.when(t < n16)
                    def _():
                        pltpu.sync_copy(
                            acc.at[pl.ds(r1 + t * L, L)],
                            nef_hbm.at[pl.ds(win0 + r1 + t * L, L)])
                    return carry
                lax.fori_loop(0, 32, f16, 0)
                r2 = r1 + lax.shift_left(n16, 4)
                nrem = limit - win0 - r2

                def f1(t, carry):
                    @pl.when(t < nrem)
                    def _():
                        pltpu.sync_copy(acc.at[pl.ds(r2 + t, 1)],
                                        nef_hbm.at[pl.ds(win0 + r2 + t, 1)])
                    return carry
                lax.fori_loop(0, L, f1, 0)
        plsc.subcore_barrier()


def kernel(node_feat, edge_index, edge_feat):
    num_nodes = node_feat.shape[0]
    cluster = jax.random.randint(jax.random.key(42), (num_nodes,), 0, N_CLUST)
    cluster = cluster.astype(jnp.int32)

    ei_flat = edge_index.reshape(-1).astype(jnp.int32)
    keys1, hist = _k1(ei_flat, cluster)
    keyp, idp = _k3(keys1, hist)
    lrank, ucounts = _k4(keyp, hist)
    srcp, dstp, nef = _k5(keyp, idp, lrank, hist, ucounts, edge_feat)
    src = srcp[:E]
    dst = dstp[:E]

    old_nodes_idx = jnp.arange(num_nodes, dtype=cluster.dtype)
    new_dst_nodes = cluster + num_nodes
    inter_src = jnp.zeros(num_nodes * 2, dtype=cluster.dtype)
    inter_src = inter_src.at[0::2].set(old_nodes_idx).at[1::2].set(new_dst_nodes)
    inter_dst = jnp.zeros(num_nodes * 2, dtype=cluster.dtype)
    inter_dst = inter_dst.at[0::2].set(new_dst_nodes).at[1::2].set(old_nodes_idx)
    cluster_score = jnp.ones((N_CLUST,), dtype=jnp.float32)
    return (src, dst, inter_src, inter_dst, cluster, nef, cluster_score)
